# Initial kernel scaffold; baseline (speedup 1.0000x reference)
import jax
import jax.numpy as jnp
from jax.experimental import pallas as pl


def kernel(item_id, cate_id, length, item_emb, cate_emb, pos_emb):
    raise NotImplementedError("write your pallas kernel here")


# speedup vs baseline: 4.6910x; 4.6910x over previous
import functools

import jax
import jax.numpy as jnp
from jax import lax
from jax.experimental import pallas as pl
from jax.experimental.pallas import tpu as pltpu
from jax.experimental.pallas import tpu_sc as plsc

NUM_CORES = 2
NUM_SUBCORES = 16
NUM_WORKERS = NUM_CORES * NUM_SUBCORES
LANES = 16


def _sc_encode(item_flat, cate_flat, item_emb, cate_emb, pos_seq, *,
               n_ids, seq_len, dim, chunk):
  per_worker = n_ids // NUM_WORKERS
  n_chunks = per_worker // chunk
  reps = chunk // seq_len

  mesh = plsc.VectorSubcoreMesh(core_axis_name="c", subcore_axis_name="s")

  @functools.partial(
      pl.kernel,
      out_type=jax.ShapeDtypeStruct((n_ids, dim), jnp.float32),
      mesh=mesh,
      scratch_types=[
          pltpu.VMEM((chunk,), jnp.int32),
          pltpu.VMEM((chunk,), jnp.int32),
          pltpu.VMEM((chunk, dim), jnp.float32),
          pltpu.VMEM((chunk, dim), jnp.float32),
          pltpu.VMEM((chunk, dim), jnp.float32),
          pltpu.SemaphoreType.DMA,
          pltpu.SemaphoreType.DMA,
      ],
      compiler_params=pltpu.CompilerParams(use_tc_tiling_on_sc=False),
  )
  def k(ii_hbm, ci_hbm, item_hbm, cate_hbm, pos_hbm, out_hbm,
        ii_v, ci_v, irow_v, crow_v, pos_v, sem_a, sem_b):
    wid = lax.axis_index("s") * NUM_CORES + lax.axis_index("c")
    wbase = wid * per_worker

    for r in range(reps):
      pltpu.sync_copy(pos_hbm, pos_v.at[pl.ds(r * seq_len, seq_len)])

    @pl.loop(0, n_chunks)
    def _(g):
      base = wbase + g * chunk
      pltpu.sync_copy(ii_hbm.at[pl.ds(base, chunk)], ii_v)
      pltpu.sync_copy(ci_hbm.at[pl.ds(base, chunk)], ci_v)
      a = pltpu.async_copy(item_hbm.at[ii_v], irow_v, sem_a)
      b = pltpu.async_copy(cate_hbm.at[ci_v], crow_v, sem_b)
      a.wait()
      b.wait()

      @pl.loop(0, chunk)
      def _(i):
        for c in range(dim // LANES):
          sl = pl.ds(c * LANES, LANES)
          irow_v[i, sl] = irow_v[i, sl] + crow_v[i, sl] + pos_v[i, sl]

      pltpu.sync_copy(irow_v, out_hbm.at[pl.ds(base, chunk)])

  return k(item_flat, cate_flat, item_emb, cate_emb, pos_seq)


def _tc_mask(length, *, batch, seq_len):
  def body(len_ref, out_ref):
    io = lax.broadcasted_iota(jnp.int32, (batch, seq_len), 1)
    out_ref[...] = io < len_ref[...]

  return pl.pallas_call(
      body,
      out_shape=jax.ShapeDtypeStruct((batch, seq_len), jnp.bool_),
  )(length)


def kernel(item_id, cate_id, length, item_emb, cate_emb, pos_emb):
  batch, seq_len = item_id.shape
  dim = item_emb.shape[1]
  n_ids = batch * seq_len
  chunk = 2 * seq_len

  item_flat = item_id.reshape(n_ids)
  cate_flat = cate_id.reshape(n_ids)
  pos_seq = pos_emb[:seq_len]

  out = _sc_encode(item_flat, cate_flat, item_emb, cate_emb, pos_seq,
                   n_ids=n_ids, seq_len=seq_len, dim=dim, chunk=chunk)
  mask = _tc_mask(length, batch=batch, seq_len=seq_len)
  return out.reshape(batch, seq_len, dim), mask


# speedup vs baseline: 5.2340x; 1.1158x over previous
import functools

import jax
import jax.numpy as jnp
from jax import lax
from jax.experimental import pallas as pl
from jax.experimental.pallas import tpu as pltpu
from jax.experimental.pallas import tpu_sc as plsc

NUM_CORES = 2
NUM_SUBCORES = 16
NUM_WORKERS = NUM_CORES * NUM_SUBCORES
LANES = 16


def _sc_encode(item_flat, cate_flat, item_emb, cate_emb, pos_seq, *,
               n_ids, seq_len, dim, chunk):
  per_worker = n_ids // NUM_WORKERS
  n_chunks = per_worker // chunk
  reps = chunk // seq_len
  assert n_chunks % 2 == 0 and reps * seq_len == chunk

  mesh = plsc.VectorSubcoreMesh(core_axis_name="c", subcore_axis_name="s")

  @functools.partial(
      pl.kernel,
      out_type=jax.ShapeDtypeStruct((n_ids, dim), jnp.float32),
      mesh=mesh,
      scratch_types=[
          pltpu.VMEM((2, chunk), jnp.int32),
          pltpu.VMEM((2, chunk), jnp.int32),
          pltpu.VMEM((2, chunk, dim), jnp.float32),
          pltpu.VMEM((2, chunk, dim), jnp.float32),
          pltpu.VMEM((seq_len, dim), jnp.float32),
          pltpu.SemaphoreType.DMA,
          pltpu.SemaphoreType.DMA,
          pltpu.SemaphoreType.DMA,
          pltpu.SemaphoreType.DMA,
      ],
      compiler_params=pltpu.CompilerParams(use_tc_tiling_on_sc=False),
  )
  def k(ii_hbm, ci_hbm, item_hbm, cate_hbm, pos_hbm, out_hbm,
        ii_v, ci_v, irow_v, crow_v, pos_v, gs0, gs1, ws0, ws1):
    gsem = (gs0, gs1)
    wsem = (ws0, ws1)
    wid = lax.axis_index("s") * NUM_CORES + lax.axis_index("c")
    wbase = wid * per_worker

    pltpu.sync_copy(pos_hbm, pos_v)

    def fetch(c, s):
      base = wbase + c * chunk
      pltpu.sync_copy(ii_hbm.at[pl.ds(base, chunk)], ii_v.at[s])
      pltpu.sync_copy(ci_hbm.at[pl.ds(base, chunk)], ci_v.at[s])
      pltpu.async_copy(item_hbm.at[ii_v.at[s]], irow_v.at[s], gsem[s])
      pltpu.async_copy(cate_hbm.at[ci_v.at[s]], crow_v.at[s], gsem[s])

    def wait_gathers(s):
      pltpu.make_async_copy(item_hbm.at[ii_v.at[s]], irow_v.at[s], gsem[s]).wait()
      pltpu.make_async_copy(cate_hbm.at[ci_v.at[s]], crow_v.at[s], gsem[s]).wait()

    def wb_desc(c, s):
      base = wbase + c * chunk
      return pltpu.make_async_copy(
          irow_v.at[s], out_hbm.at[pl.ds(base, chunk)], wsem[s])

    fetch(0, 0)

    @pl.loop(0, n_chunks, step=2)
    def _(g):
      for s in range(2):
        c = g + s
        sn = 1 - s

        @pl.when(c >= 1)
        def _():
          wb_desc(c - 1, sn).wait()

        @pl.when(c + 1 < n_chunks)
        def _():
          fetch(c + 1, sn)

        wait_gathers(s)

        for r in range(reps):
          off = r * seq_len

          @pl.loop(0, seq_len)
          def _(l):
            i = off + l
            for cc in range(dim // LANES):
              sl = pl.ds(cc * LANES, LANES)
              irow_v[s, i, sl] = irow_v[s, i, sl] + crow_v[s, i, sl] + pos_v[l, sl]

        wb_desc(c, s).start()

    wb_desc(n_chunks - 1, (n_chunks - 1) % 2).wait()

  return k(item_flat, cate_flat, item_emb, cate_emb, pos_seq)


def _tc_mask(length, *, batch, seq_len):
  def body(len_ref, out_ref):
    io = lax.broadcasted_iota(jnp.int32, (batch, seq_len), 1)
    out_ref[...] = io < len_ref[...]

  return pl.pallas_call(
      body,
      out_shape=jax.ShapeDtypeStruct((batch, seq_len), jnp.bool_),
  )(length)


def kernel(item_id, cate_id, length, item_emb, cate_emb, pos_emb):
  batch, seq_len = item_id.shape
  dim = item_emb.shape[1]
  n_ids = batch * seq_len
  chunk = 4 * seq_len

  item_flat = item_id.reshape(n_ids)
  cate_flat = cate_id.reshape(n_ids)
  pos_seq = pos_emb[:seq_len]

  out = _sc_encode(item_flat, cate_flat, item_emb, cate_emb, pos_seq,
                   n_ids=n_ids, seq_len=seq_len, dim=dim, chunk=chunk)
  mask = _tc_mask(length, batch=batch, seq_len=seq_len)
  return out.reshape(batch, seq_len, dim), mask


# speedup vs baseline: 5.4198x; 1.0355x over previous
import functools

import jax
import jax.numpy as jnp
from jax import lax
from jax.experimental import pallas as pl
from jax.experimental.pallas import tpu as pltpu
from jax.experimental.pallas import tpu_sc as plsc

NUM_CORES = 2
NUM_SUBCORES = 16
NUM_WORKERS = NUM_CORES * NUM_SUBCORES
LANES = 16


def _sc_encode(item_flat, cate_flat, item_emb, cate_emb, pos_seq, *,
               n_ids, seq_len, batch, dim, chunk):
  per_worker = n_ids // NUM_WORKERS
  n_chunks = per_worker // chunk
  assert n_chunks % 2 == 0 and batch % chunk == 0 and per_worker % chunk == 0

  mesh = plsc.VectorSubcoreMesh(core_axis_name="c", subcore_axis_name="s")

  @functools.partial(
      pl.kernel,
      out_type=jax.ShapeDtypeStruct((n_ids, dim), jnp.float32),
      mesh=mesh,
      scratch_types=[
          pltpu.VMEM((2, chunk), jnp.int32),
          pltpu.VMEM((2, chunk), jnp.int32),
          pltpu.VMEM((2, chunk, dim), jnp.float32),
          pltpu.VMEM((2, chunk, dim), jnp.float32),
          pltpu.VMEM((seq_len, dim), jnp.float32),
          pltpu.SemaphoreType.DMA,
          pltpu.SemaphoreType.DMA,
          pltpu.SemaphoreType.DMA,
          pltpu.SemaphoreType.DMA,
      ],
      compiler_params=pltpu.CompilerParams(use_tc_tiling_on_sc=False),
  )
  def k(ii_hbm, ci_hbm, item_hbm, cate_hbm, pos_hbm, out_hbm,
        ii_v, ci_v, irow_v, crow_v, pos_v, gs0, gs1, ws0, ws1):
    gsem = (gs0, gs1)
    wsem = (ws0, ws1)
    wid = lax.axis_index("s") * NUM_CORES + lax.axis_index("c")
    wbase = wid * per_worker

    pltpu.sync_copy(pos_hbm, pos_v)

    def fetch(c, s):
      base = wbase + c * chunk
      pltpu.sync_copy(ii_hbm.at[pl.ds(base, chunk)], ii_v.at[s])
      pltpu.sync_copy(ci_hbm.at[pl.ds(base, chunk)], ci_v.at[s])
      pltpu.async_copy(item_hbm.at[ii_v.at[s]], irow_v.at[s], gsem[s])
      pltpu.async_copy(cate_hbm.at[ci_v.at[s]], crow_v.at[s], gsem[s])

    def wait_gathers(s):
      pltpu.make_async_copy(item_hbm.at[ii_v.at[s]], irow_v.at[s], gsem[s]).wait()
      pltpu.make_async_copy(cate_hbm.at[ci_v.at[s]], crow_v.at[s], gsem[s]).wait()

    def wb_desc(c, s):
      base = wbase + c * chunk
      return pltpu.make_async_copy(
          irow_v.at[s], out_hbm.at[pl.ds(base, chunk)], wsem[s])

    fetch(0, 0)

    @pl.loop(0, n_chunks, step=2)
    def _(g):
      for s in range(2):
        c = g + s
        sn = 1 - s

        @pl.when(c >= 1)
        def _():
          wb_desc(c - 1, sn).wait()

        @pl.when(c + 1 < n_chunks)
        def _():
          fetch(c + 1, sn)

        wait_gathers(s)

        lg = (wbase + c * chunk) // batch
        prow = [pos_v[lg, pl.ds(cc * LANES, LANES)] for cc in range(dim // LANES)]

        @pl.loop(0, chunk)
        def _(i):
          for cc in range(dim // LANES):
            sl = pl.ds(cc * LANES, LANES)
            irow_v[s, i, sl] = irow_v[s, i, sl] + crow_v[s, i, sl] + prow[cc]

        wb_desc(c, s).start()

    wb_desc(n_chunks - 1, (n_chunks - 1) % 2).wait()

  return k(item_flat, cate_flat, item_emb, cate_emb, pos_seq)


def _tc_mask(length, *, batch, seq_len):
  def body(len_ref, out_ref):
    io = lax.broadcasted_iota(jnp.int32, (batch, seq_len), 1)
    out_ref[...] = io < len_ref[...]

  return pl.pallas_call(
      body,
      out_shape=jax.ShapeDtypeStruct((batch, seq_len), jnp.bool_),
  )(length)


def kernel(item_id, cate_id, length, item_emb, cate_emb, pos_emb):
  batch, seq_len = item_id.shape
  dim = item_emb.shape[1]
  n_ids = batch * seq_len
  chunk = 512

  item_flat = item_id.T.reshape(n_ids)
  cate_flat = cate_id.T.reshape(n_ids)
  pos_seq = pos_emb[:seq_len]

  out = _sc_encode(item_flat, cate_flat, item_emb, cate_emb, pos_seq,
                   n_ids=n_ids, seq_len=seq_len, batch=batch, dim=dim,
                   chunk=chunk)
  seq = jnp.transpose(out.reshape(seq_len, batch, dim), (1, 0, 2))
  mask = _tc_mask(length, batch=batch, seq_len=seq_len)
  return seq, mask


# speedup vs baseline: 5.5011x; 1.0150x over previous
import functools

import jax
import jax.numpy as jnp
from jax import lax
from jax.experimental import pallas as pl
from jax.experimental.pallas import tpu as pltpu
from jax.experimental.pallas import tpu_sc as plsc

NUM_CORES = 2
NUM_SUBCORES = 16
NUM_WORKERS = NUM_CORES * NUM_SUBCORES
LANES = 16


def _sc_encode(item_flat, cate_flat, item_emb, cate_emb, pos_seq, *,
               n_ids, seq_len, batch, dim, chunk):
  per_worker = n_ids // NUM_WORKERS
  n_chunks = per_worker // chunk
  assert n_chunks % 2 == 0 and batch % chunk == 0 and per_worker % chunk == 0
  assert n_ids * dim % 128 == 0

  mesh = plsc.VectorSubcoreMesh(core_axis_name="c", subcore_axis_name="s")

  @functools.partial(
      pl.kernel,
      out_type=jax.ShapeDtypeStruct((n_ids * dim // 128, 128), jnp.float32),
      mesh=mesh,
      scratch_types=[
          pltpu.VMEM((2, chunk), jnp.int32),
          pltpu.VMEM((2, chunk), jnp.int32),
          pltpu.VMEM((2, chunk, dim), jnp.float32),
          pltpu.VMEM((2, chunk, dim), jnp.float32),
          pltpu.VMEM((2, chunk * dim // 128, 128), jnp.float32),
          pltpu.VMEM((seq_len, dim), jnp.float32),
          pltpu.SemaphoreType.DMA,
          pltpu.SemaphoreType.DMA,
          pltpu.SemaphoreType.DMA,
          pltpu.SemaphoreType.DMA,
      ],
      compiler_params=pltpu.CompilerParams(use_tc_tiling_on_sc=False),
  )
  def k(ii_hbm, ci_hbm, item_hbm, cate_hbm, pos_hbm, out128_hbm,
        ii_v, ci_v, irow_v, crow_v, out_v, pos_v, gs0, gs1, ws0, ws1):
    gsem = (gs0, gs1)
    wsem = (ws0, ws1)
    wid = lax.axis_index("s") * NUM_CORES + lax.axis_index("c")
    wbase = wid * per_worker

    pltpu.sync_copy(pos_hbm, pos_v)

    def fetch(c, s):
      base = wbase + c * chunk
      pltpu.sync_copy(ii_hbm.at[pl.ds(base, chunk)], ii_v.at[s])
      pltpu.sync_copy(ci_hbm.at[pl.ds(base, chunk)], ci_v.at[s])
      pltpu.async_copy(item_hbm.at[ii_v.at[s]], irow_v.at[s], gsem[s])
      pltpu.async_copy(cate_hbm.at[ci_v.at[s]], crow_v.at[s], gsem[s])

    def wait_gathers(s):
      pltpu.make_async_copy(item_hbm.at[ii_v.at[s]], irow_v.at[s], gsem[s]).wait()
      pltpu.make_async_copy(cate_hbm.at[ci_v.at[s]], crow_v.at[s], gsem[s]).wait()

    fold = chunk * dim // 128

    def wb_desc(c, s):
      base = (wbase + c * chunk) * dim // 128
      return pltpu.make_async_copy(
          out_v.at[s], out128_hbm.at[pl.ds(base, fold)], wsem[s])

    fetch(0, 0)

    @pl.loop(0, n_chunks, step=2)
    def _(g):
      for s in range(2):
        c = g + s
        sn = 1 - s

        @pl.when(c >= 1)
        def _():
          wb_desc(c - 1, sn).wait()

        @pl.when(c + 1 < n_chunks)
        def _():
          fetch(c + 1, sn)

        wait_gathers(s)

        lg = (wbase + c * chunk) // batch
        prow = [pos_v[lg, pl.ds(cc * LANES, LANES)] for cc in range(dim // LANES)]
        per_row = 128 // dim

        @pl.loop(0, fold)
        def _(j):
          i0 = j * per_row
          for q in range(per_row):
            for cc in range(dim // LANES):
              src = pl.ds(cc * LANES, LANES)
              dst = pl.ds(q * dim + cc * LANES, LANES)
              out_v[s, j, dst] = (irow_v[s, i0 + q, src]
                                  + crow_v[s, i0 + q, src] + prow[cc])

        wb_desc(c, s).start()

    wb_desc(n_chunks - 1, (n_chunks - 1) % 2).wait()

  return k(item_flat, cate_flat, item_emb, cate_emb, pos_seq)


def _tc_mask(length, *, batch, seq_len):
  def body(len_ref, out_ref):
    io = lax.broadcasted_iota(jnp.int32, (batch, seq_len), 1)
    out_ref[...] = io < len_ref[...]

  return pl.pallas_call(
      body,
      out_shape=jax.ShapeDtypeStruct((batch, seq_len), jnp.bool_),
  )(length)


def kernel(item_id, cate_id, length, item_emb, cate_emb, pos_emb):
  batch, seq_len = item_id.shape
  dim = item_emb.shape[1]
  n_ids = batch * seq_len
  chunk = 512

  item_flat = item_id.T.reshape(n_ids)
  cate_flat = cate_id.T.reshape(n_ids)
  pos_seq = pos_emb[:seq_len]

  out = _sc_encode(item_flat, cate_flat, item_emb, cate_emb, pos_seq,
                   n_ids=n_ids, seq_len=seq_len, batch=batch, dim=dim,
                   chunk=chunk)
  seq = jnp.transpose(out.reshape(seq_len, batch, dim), (1, 0, 2))
  mask = _tc_mask(length, batch=batch, seq_len=seq_len)
  return seq, mask
